# baseline (speedup 1.0000x reference)
import functools

import jax
import jax.numpy as jnp
from jax import lax
from jax.experimental import pallas as pl
from jax.experimental.pallas import tpu as pltpu
from jax.experimental.pallas import tpu_sc as plsc

N = 10000
E = 320000
H = 128
NC = 2
NS = 16
NW = NC * NS
CH = 64
NSLOT = 4
NCH = 156
NREM = (E - NW * NCH * CH) // CH
NP = 10240
RPT = NP // NS


def _sc_scatter_body(ea_hbm, ei_hbm, out_hbm, ibuf, dbuf, acc_sh,
                     dsems, isems):
    c = lax.axis_index("c")
    s = lax.axis_index("s")
    wid = s * NC + c
    gb = wid * NCH

    def dsrc(g):
        return ea_hbm.at[0, pl.ds(g * CH, CH), :]

    def isrc(g):
        return ei_hbm.at[0, 0, pl.ds(g * CH, CH)]

    for b in range(1, NSLOT):
        pltpu.async_copy(dsrc(gb + b), dbuf.at[b], dsems.at[b])
        pltpu.async_copy(isrc(gb + b), ibuf.at[b], isems.at[b])
    pltpu.async_copy(isrc(gb), ibuf.at[0], isems.at[0])

    def zstore(i, _):
        dbuf[0, i // 8, pl.ds((i % 8) * 16, 16)] = jnp.zeros((16,), jnp.float32)
        return 0
    lax.fori_loop(0, CH * 8, zstore, 0)
    for j in range(RPT // CH):
        pltpu.sync_copy(dbuf.at[0], acc_sh.at[pl.ds(s * RPT + j * CH, CH), :])
    plsc.subcore_barrier()

    pltpu.async_copy(dsrc(gb), dbuf.at[0], dsems.at[0])

    def quad_body(i, _):
        for b in range(NSLOT):
            g = gb + NSLOT * i + b
            pltpu.make_async_copy(dsrc(g), dbuf.at[b], dsems.at[b]).wait()
            pltpu.make_async_copy(isrc(g), ibuf.at[b], isems.at[b]).wait()
            pltpu.sync_copy(dbuf.at[b], acc_sh.at[ibuf.at[b]], add=True)

            @pl.when(NSLOT * i + b + NSLOT < NCH)
            def _():
                pltpu.async_copy(dsrc(g + NSLOT), dbuf.at[b], dsems.at[b])
                pltpu.async_copy(isrc(g + NSLOT), ibuf.at[b], isems.at[b])
        return 0
    lax.fori_loop(0, NCH // NSLOT, quad_body, 0)

    @pl.when(wid < NREM)
    def _():
        g = NW * NCH + wid
        pltpu.async_copy(dsrc(g), dbuf.at[0], dsems.at[0])
        pltpu.async_copy(isrc(g), ibuf.at[0], isems.at[0])
        pltpu.make_async_copy(dsrc(g), dbuf.at[0], dsems.at[0]).wait()
        pltpu.make_async_copy(isrc(g), ibuf.at[0], isems.at[0]).wait()
        pltpu.sync_copy(dbuf.at[0], acc_sh.at[ibuf.at[0]], add=True)

    plsc.subcore_barrier()

    pltpu.sync_copy(acc_sh.at[pl.ds(s * RPT, RPT), :],
                    out_hbm.at[c, pl.ds(s * RPT, RPT), :])


@functools.partial(
    pl.kernel,
    out_type=jax.ShapeDtypeStruct((NC, NP, H), jnp.float32),
    mesh=plsc.VectorSubcoreMesh(core_axis_name="c", subcore_axis_name="s"),
    scratch_types=[
        pltpu.VMEM((NSLOT, CH), jnp.int32),
        pltpu.VMEM((NSLOT, CH, H), jnp.float32),
        pltpu.VMEM_SHARED((NP, H), jnp.float32),
        pltpu.SemaphoreType.DMA((NSLOT,)),
        pltpu.SemaphoreType.DMA((NSLOT,)),
    ],
)
def _sc_scatter(ea_hbm, ei_hbm, out_hbm, ibuf, dbuf, acc_sh, dsems, isems):
    _sc_scatter_body(ea_hbm, ei_hbm, out_hbm, ibuf, dbuf, acc_sh,
                     dsems, isems)


BN = 400


def _mlp_body(x_ref, p_ref, w1x_ref, w1e_ref, b1_ref, w2_ref,
              b2_ref, g_ref, bt_ref, o_ref):
    xb = x_ref[0]
    sb = p_ref[0] + p_ref[1]
    h = jnp.dot(xb, w1x_ref[...], preferred_element_type=jnp.float32)
    h = h + jnp.dot(sb, w1e_ref[...], preferred_element_type=jnp.float32)
    h = jnp.maximum(h + b1_ref[...], 0.0)
    o = jnp.dot(h, w2_ref[...], preferred_element_type=jnp.float32)
    o = o + b2_ref[...] + xb
    mu = jnp.mean(o, axis=-1, keepdims=True)
    d = o - mu
    var = jnp.mean(d * d, axis=-1, keepdims=True)
    o_ref[0] = d * lax.rsqrt(var + 1e-5) * g_ref[...] + bt_ref[...]


def _mlp(x, partial, w1x, w1e, b1, w2, b2, g, bt):
    full = pl.BlockSpec((H, H), lambda i: (0, 0))
    vec = pl.BlockSpec((1, H), lambda i: (0, 0))
    xrows = pl.BlockSpec((1, BN, H), lambda i: (0, i, 0))
    prows = pl.BlockSpec((2, BN, H), lambda i: (0, i, 0))
    return pl.pallas_call(
        _mlp_body,
        grid=(N // BN,),
        in_specs=[xrows, prows, full, full, vec, full, vec, vec, vec],
        out_specs=xrows,
        out_shape=jax.ShapeDtypeStruct((1, N, H), jnp.float32),
    )(x, partial, w1x, w1e, b1, w2, b2, g, bt)


def kernel(x, edge_index, edge_attr, W1, b1, W2, b2, gamma, beta):
    partial = _sc_scatter(edge_attr, edge_index)
    return _mlp(x, partial, W1[:H], W1[H:],
                b1.reshape(1, H), W2, b2.reshape(1, H),
                gamma.reshape(1, H), beta.reshape(1, H))


# speedup vs baseline: 1.1020x; 1.1020x over previous
import functools

import jax
import jax.numpy as jnp
from jax import lax
from jax.experimental import pallas as pl
from jax.experimental.pallas import tpu as pltpu
from jax.experimental.pallas import tpu_sc as plsc

N = 10000
E = 320000
H = 128
NC = 2
NS = 16
NW = NC * NS
CH = 64
NSLOT = 4
NCH = 156
NREM = (E - NW * NCH * CH) // CH
NP = 10240
RPT = NP // NS


def _sc_scatter_body(ea_hbm, ei_hbm, out_hbm, ibuf, dbuf, acc_sh,
                     dsems, isems):
    c = lax.axis_index("c")
    s = lax.axis_index("s")
    wid = s * NC + c
    gb = wid * NCH

    def dsrc(g):
        return ea_hbm.at[0, pl.ds(g * CH, CH), :]

    def isrc(g):
        return ei_hbm.at[0, 0, pl.ds(g * CH, CH)]

    for b in range(1, NSLOT):
        pltpu.async_copy(dsrc(gb + b), dbuf.at[b], dsems.at[b])
        pltpu.async_copy(isrc(gb + b), ibuf.at[b], isems.at[b])
    pltpu.async_copy(isrc(gb), ibuf.at[0], isems.at[0])

    def zstore(i, _):
        dbuf[0, i // 8, pl.ds((i % 8) * 16, 16)] = jnp.zeros((16,), jnp.float32)
        return 0
    lax.fori_loop(0, CH * 8, zstore, 0)
    for j in range(RPT // CH):
        pltpu.sync_copy(dbuf.at[0], acc_sh.at[pl.ds(s * RPT + j * CH, CH), :])
    plsc.subcore_barrier()

    pltpu.async_copy(dsrc(gb), dbuf.at[0], dsems.at[0])

    def quad_body(i, _):
        for b in range(NSLOT):
            g = gb + NSLOT * i + b
            pltpu.make_async_copy(dsrc(g), dbuf.at[b], dsems.at[b]).wait()
            pltpu.make_async_copy(isrc(g), ibuf.at[b], isems.at[b]).wait()
            pltpu.sync_copy(dbuf.at[b], acc_sh.at[ibuf.at[b]], add=True)

            @pl.when(NSLOT * i + b + NSLOT < NCH)
            def _():
                pltpu.async_copy(dsrc(g + NSLOT), dbuf.at[b], dsems.at[b])
                pltpu.async_copy(isrc(g + NSLOT), ibuf.at[b], isems.at[b])
        return 0
    lax.fori_loop(0, NCH // NSLOT, quad_body, 0)

    @pl.when(wid < NREM)
    def _():
        g = NW * NCH + wid
        pltpu.async_copy(dsrc(g), dbuf.at[0], dsems.at[0])
        pltpu.async_copy(isrc(g), ibuf.at[0], isems.at[0])
        pltpu.make_async_copy(dsrc(g), dbuf.at[0], dsems.at[0]).wait()
        pltpu.make_async_copy(isrc(g), ibuf.at[0], isems.at[0]).wait()
        pltpu.sync_copy(dbuf.at[0], acc_sh.at[ibuf.at[0]], add=True)

    plsc.subcore_barrier()

    pltpu.sync_copy(acc_sh.at[pl.ds(s * RPT, RPT), :],
                    out_hbm.at[c, pl.ds(s * RPT, RPT), :])


@functools.partial(
    pl.kernel,
    out_type=jax.ShapeDtypeStruct((NC, NP, H), jnp.float32),
    mesh=plsc.VectorSubcoreMesh(core_axis_name="c", subcore_axis_name="s"),
    scratch_types=[
        pltpu.VMEM((NSLOT, CH), jnp.int32),
        pltpu.VMEM((NSLOT, CH, H), jnp.float32),
        pltpu.VMEM_SHARED((NP, H), jnp.float32),
        pltpu.SemaphoreType.DMA((NSLOT,)),
        pltpu.SemaphoreType.DMA((NSLOT,)),
    ],
)
def _sc_scatter(ea_hbm, ei_hbm, out_hbm, ibuf, dbuf, acc_sh, dsems, isems):
    _sc_scatter_body(ea_hbm, ei_hbm, out_hbm, ibuf, dbuf, acc_sh,
                     dsems, isems)


BN = 2000


def _mlp_body(x_ref, p_ref, w1x_ref, w1e_ref, b1_ref, w2_ref,
              b2_ref, g_ref, bt_ref, o_ref):
    xb = x_ref[0]
    sb = p_ref[0] + p_ref[1]
    h = jnp.dot(xb, w1x_ref[...], preferred_element_type=jnp.float32)
    h = h + jnp.dot(sb, w1e_ref[...], preferred_element_type=jnp.float32)
    h = jnp.maximum(h + b1_ref[...], 0.0)
    o = jnp.dot(h, w2_ref[...], preferred_element_type=jnp.float32)
    o = o + b2_ref[...] + xb
    mu = jnp.mean(o, axis=-1, keepdims=True)
    d = o - mu
    var = jnp.mean(d * d, axis=-1, keepdims=True)
    o_ref[0] = d * lax.rsqrt(var + 1e-5) * g_ref[...] + bt_ref[...]


def _mlp(x, partial, w1x, w1e, b1, w2, b2, g, bt):
    full = pl.BlockSpec((H, H), lambda i: (0, 0))
    vec = pl.BlockSpec((1, H), lambda i: (0, 0))
    xrows = pl.BlockSpec((1, BN, H), lambda i: (0, i, 0))
    prows = pl.BlockSpec((2, BN, H), lambda i: (0, i, 0))
    return pl.pallas_call(
        _mlp_body,
        grid=(N // BN,),
        in_specs=[xrows, prows, full, full, vec, full, vec, vec, vec],
        out_specs=xrows,
        out_shape=jax.ShapeDtypeStruct((1, N, H), jnp.float32),
    )(x, partial, w1x, w1e, b1, w2, b2, g, bt)


def kernel(x, edge_index, edge_attr, W1, b1, W2, b2, gamma, beta):
    partial = _sc_scatter(edge_attr, edge_index)
    return _mlp(x, partial, W1[:H], W1[H:],
                b1.reshape(1, H), W2, b2.reshape(1, H),
                gamma.reshape(1, H), beta.reshape(1, H))


# speedup vs baseline: 1.1021x; 1.0001x over previous
import functools

import jax
import jax.numpy as jnp
from jax import lax
from jax.experimental import pallas as pl
from jax.experimental.pallas import tpu as pltpu
from jax.experimental.pallas import tpu_sc as plsc

N = 10000
E = 320000
H = 128
NC = 2
NS = 16
NW = NC * NS
CH = 64
NSLOT = 5
NCH = 156
NREM = (E - NW * NCH * CH) // CH
NP = 10240
RPT = NP // NS


def _sc_scatter_body(ea_hbm, ei_hbm, out_hbm, ibuf, dbuf, acc_sh,
                     dsems, isems):
    c = lax.axis_index("c")
    s = lax.axis_index("s")
    wid = s * NC + c
    gb = wid * NCH

    def dsrc(g):
        return ea_hbm.at[0, pl.ds(g * CH, CH), :]

    def isrc(g):
        return ei_hbm.at[0, 0, pl.ds(g * CH, CH)]

    for b in range(1, NSLOT):
        pltpu.async_copy(dsrc(gb + b), dbuf.at[b], dsems.at[b])
        pltpu.async_copy(isrc(gb + b), ibuf.at[b], isems.at[b])
    pltpu.async_copy(isrc(gb), ibuf.at[0], isems.at[0])

    def zstore(i, _):
        dbuf[0, i // 8, pl.ds((i % 8) * 16, 16)] = jnp.zeros((16,), jnp.float32)
        return 0
    lax.fori_loop(0, CH * 8, zstore, 0)
    for j in range(RPT // CH):
        pltpu.sync_copy(dbuf.at[0], acc_sh.at[pl.ds(s * RPT + j * CH, CH), :])
    plsc.subcore_barrier()

    pltpu.async_copy(dsrc(gb), dbuf.at[0], dsems.at[0])

    def quad_body(i, _):
        for b in range(NSLOT):
            g = gb + NSLOT * i + b
            pltpu.make_async_copy(dsrc(g), dbuf.at[b], dsems.at[b]).wait()
            pltpu.make_async_copy(isrc(g), ibuf.at[b], isems.at[b]).wait()
            pltpu.sync_copy(dbuf.at[b], acc_sh.at[ibuf.at[b]], add=True)

            @pl.when(NSLOT * i + b + NSLOT < NCH)
            def _():
                pltpu.async_copy(dsrc(g + NSLOT), dbuf.at[b], dsems.at[b])
                pltpu.async_copy(isrc(g + NSLOT), ibuf.at[b], isems.at[b])
        return 0
    lax.fori_loop(0, (NCH - 1) // NSLOT, quad_body, 0)

    gl = gb + NCH - 1
    pltpu.make_async_copy(dsrc(gl), dbuf.at[0], dsems.at[0]).wait()
    pltpu.make_async_copy(isrc(gl), ibuf.at[0], isems.at[0]).wait()
    pltpu.sync_copy(dbuf.at[0], acc_sh.at[ibuf.at[0]], add=True)

    @pl.when(wid < NREM)
    def _():
        g = NW * NCH + wid
        pltpu.async_copy(dsrc(g), dbuf.at[0], dsems.at[0])
        pltpu.async_copy(isrc(g), ibuf.at[0], isems.at[0])
        pltpu.make_async_copy(dsrc(g), dbuf.at[0], dsems.at[0]).wait()
        pltpu.make_async_copy(isrc(g), ibuf.at[0], isems.at[0]).wait()
        pltpu.sync_copy(dbuf.at[0], acc_sh.at[ibuf.at[0]], add=True)

    plsc.subcore_barrier()

    pltpu.sync_copy(acc_sh.at[pl.ds(s * RPT, RPT), :],
                    out_hbm.at[c, pl.ds(s * RPT, RPT), :])


@functools.partial(
    pl.kernel,
    out_type=jax.ShapeDtypeStruct((NC, NP, H), jnp.float32),
    mesh=plsc.VectorSubcoreMesh(core_axis_name="c", subcore_axis_name="s"),
    scratch_types=[
        pltpu.VMEM((NSLOT, CH), jnp.int32),
        pltpu.VMEM((NSLOT, CH, H), jnp.float32),
        pltpu.VMEM_SHARED((NP, H), jnp.float32),
        pltpu.SemaphoreType.DMA((NSLOT,)),
        pltpu.SemaphoreType.DMA((NSLOT,)),
    ],
)
def _sc_scatter(ea_hbm, ei_hbm, out_hbm, ibuf, dbuf, acc_sh, dsems, isems):
    _sc_scatter_body(ea_hbm, ei_hbm, out_hbm, ibuf, dbuf, acc_sh,
                     dsems, isems)


BN = 2000


def _mlp_body(x_ref, p_ref, w1x_ref, w1e_ref, b1_ref, w2_ref,
              b2_ref, g_ref, bt_ref, o_ref):
    xb = x_ref[0]
    sb = p_ref[0] + p_ref[1]
    h = jnp.dot(xb, w1x_ref[...], preferred_element_type=jnp.float32)
    h = h + jnp.dot(sb, w1e_ref[...], preferred_element_type=jnp.float32)
    h = jnp.maximum(h + b1_ref[...], 0.0)
    o = jnp.dot(h, w2_ref[...], preferred_element_type=jnp.float32)
    o = o + b2_ref[...] + xb
    mu = jnp.mean(o, axis=-1, keepdims=True)
    d = o - mu
    var = jnp.mean(d * d, axis=-1, keepdims=True)
    o_ref[0] = d * lax.rsqrt(var + 1e-5) * g_ref[...] + bt_ref[...]


def _mlp(x, partial, w1x, w1e, b1, w2, b2, g, bt):
    full = pl.BlockSpec((H, H), lambda i: (0, 0))
    vec = pl.BlockSpec((1, H), lambda i: (0, 0))
    xrows = pl.BlockSpec((1, BN, H), lambda i: (0, i, 0))
    prows = pl.BlockSpec((2, BN, H), lambda i: (0, i, 0))
    return pl.pallas_call(
        _mlp_body,
        grid=(N // BN,),
        in_specs=[xrows, prows, full, full, vec, full, vec, vec, vec],
        out_specs=xrows,
        out_shape=jax.ShapeDtypeStruct((1, N, H), jnp.float32),
    )(x, partial, w1x, w1e, b1, w2, b2, g, bt)


def kernel(x, edge_index, edge_attr, W1, b1, W2, b2, gamma, beta):
    partial = _sc_scatter(edge_attr, edge_index)
    return _mlp(x, partial, W1[:H], W1[H:],
                b1.reshape(1, H), W2, b2.reshape(1, H),
                gamma.reshape(1, H), beta.reshape(1, H))
